# Initial kernel scaffold; baseline (speedup 1.0000x reference)
import functools

import jax
import jax.numpy as jnp
from jax import lax
from jax.experimental import pallas as pl
from jax.experimental.pallas import tpu as pltpu
from jax.experimental.pallas import tpu_sc as plsc

N = 10000
E = 320000
H = 128
R = 4

NC = 2
NS = 16
NW = NC * NS
CH = 128
EPW = ((E + NW * CH - 1) // (NW * CH)) * CH
EP = EPW * NW
PROWS = EP // 128
RB = 1000
RPT = N // NS


def _ymm_body(x_ref, w_ref, y_ref):
    y_ref[0] = jnp.dot(x_ref[...], w_ref[0], preferred_element_type=jnp.float32)


def _prep_body(relp_ref, src_ref, w_ref, gidx_ref, wm_ref):
    r0 = lax.broadcasted_iota(jnp.int32, (PROWS, 128), 0)
    c0 = lax.broadcasted_iota(jnp.int32, (PROWS, 128), 1)
    i = r0 * 128 + c0
    p0, p1, p2, p3, p4 = (relp_ref[k] for k in range(5))
    rel = ((i >= p1).astype(jnp.int32) + (i >= p2).astype(jnp.int32)
           + (i >= p3).astype(jnp.int32))
    valid = (i >= p0) & (i < p4)
    gidx_ref[...] = src_ref[...] + N * rel
    wm_ref[...] = w_ref[...] * valid.astype(jnp.float32)


def _comb_body(relp_ref, x_ref, ws_ref, bs_ref, p_ref, o_ref):
    act = [jnp.where(relp_ref[r + 1] > relp_ref[r], 1.0, 0.0).astype(jnp.float32)
           for r in range(R)]
    w_sum = (act[0] * ws_ref[0] + act[1] * ws_ref[1]
             + act[2] * ws_ref[2] + act[3] * ws_ref[3])
    b_sum = (act[0] * bs_ref[0] + act[1] * bs_ref[1]
             + act[2] * bs_ref[2] + act[3] * bs_ref[3])
    o_ref[...] = (jnp.dot(x_ref[...], w_sum, preferred_element_type=jnp.float32)
                  + b_sum[None, :] + p_ref[0] + p_ref[1])


def _sc_scatter(y_hbm, gidx_hbm, dst_hbm, w_hbm, z_hbm, out_hbm,
                acc, idx_v, dst_v, w_v, rows_v, sem):
    cid = lax.axis_index("c")
    sid = lax.axis_index("s")
    wid = sid * NC + cid
    pltpu.sync_copy(z_hbm, acc.at[pl.ds(sid * RPT, RPT)])
    plsc.subcore_barrier()

    ebase = wid * EPW

    def row(k, _):
        wk = plsc.load_gather(w_v, [jnp.full((16,), k, jnp.int32)])
        for j in range(H // 16):
            sl = pl.ds(j * 16, 16)
            rows_v[k, sl] = rows_v[k, sl] * wk
        return 0

    def chunk(c, _):
        b = pl.multiple_of(ebase + c * CH, CH)
        pltpu.sync_copy(gidx_hbm.at[pl.ds(b, CH)], idx_v)
        pltpu.sync_copy(dst_hbm.at[pl.ds(b, CH)], dst_v)
        pltpu.sync_copy(w_hbm.at[pl.ds(b, CH)], w_v)
        pltpu.async_copy(y_hbm.at[idx_v], rows_v, sem).wait()
        lax.fori_loop(0, CH, row, 0)
        pltpu.sync_copy(rows_v, acc.at[dst_v], add=True)
        return 0

    lax.fori_loop(0, EPW // CH, chunk, 0)
    plsc.subcore_barrier()
    pltpu.sync_copy(acc.at[pl.ds(sid * RPT, RPT)],
                    out_hbm.at[cid, pl.ds(sid * RPT, RPT)])


def kernel(x, edge_index, edge_weight, rel_ptr, W_self, b_self, W_nei):
    x = x.astype(jnp.float32)
    src = edge_index[0].astype(jnp.int32)
    dst = edge_index[1].astype(jnp.int32)
    w = edge_weight.astype(jnp.float32)
    relp = rel_ptr.astype(jnp.int32)

    pad = EP - E
    src2 = jnp.pad(src, (0, pad)).reshape(PROWS, 128)
    w2 = jnp.pad(w, (0, pad)).reshape(PROWS, 128)
    dst_p = jnp.pad(dst, (0, pad))

    y = pl.pallas_call(
        _ymm_body,
        grid=(R, N // RB),
        in_specs=[
            pl.BlockSpec((RB, H), lambda r, i: (i, 0)),
            pl.BlockSpec((1, H, H), lambda r, i: (r, 0, 0)),
        ],
        out_specs=pl.BlockSpec((1, RB, H), lambda r, i: (r, i, 0)),
        out_shape=jax.ShapeDtypeStruct((R, N, H), jnp.float32),
    )(x, W_nei.astype(jnp.float32))
    y_flat = y.reshape(R * N, H)

    gidx2, wm2 = pl.pallas_call(
        _prep_body,
        in_specs=[
            pl.BlockSpec(memory_space=pltpu.SMEM),
            pl.BlockSpec((PROWS, 128), lambda: (0, 0)),
            pl.BlockSpec((PROWS, 128), lambda: (0, 0)),
        ],
        out_specs=[
            pl.BlockSpec((PROWS, 128), lambda: (0, 0)),
            pl.BlockSpec((PROWS, 128), lambda: (0, 0)),
        ],
        out_shape=[
            jax.ShapeDtypeStruct((PROWS, 128), jnp.int32),
            jax.ShapeDtypeStruct((PROWS, 128), jnp.float32),
        ],
    )(relp, src2, w2)
    gidx = gidx2.reshape(EP)
    wm = wm2.reshape(EP)

    zrows = jnp.zeros((RPT, H), jnp.float32)
    mesh = plsc.VectorSubcoreMesh(core_axis_name="c", subcore_axis_name="s",
                                  num_cores=NC, num_subcores=NS)
    part = functools.partial(
        pl.kernel,
        out_type=jax.ShapeDtypeStruct((NC, N, H), jnp.float32),
        mesh=mesh,
        scratch_types=[
            pltpu.VMEM_SHARED((N, H), jnp.float32),
            pltpu.VMEM((CH,), jnp.int32),
            pltpu.VMEM((CH,), jnp.int32),
            pltpu.VMEM((CH,), jnp.float32),
            pltpu.VMEM((CH, H), jnp.float32),
            pltpu.SemaphoreType.DMA,
        ],
    )(_sc_scatter)(y_flat, gidx, dst_p, wm, zrows)

    out = pl.pallas_call(
        _comb_body,
        grid=(N // RB,),
        in_specs=[
            pl.BlockSpec(memory_space=pltpu.SMEM),
            pl.BlockSpec((RB, H), lambda i: (i, 0)),
            pl.BlockSpec((R, H, H), lambda i: (0, 0, 0)),
            pl.BlockSpec((R, H), lambda i: (0, 0)),
            pl.BlockSpec((NC, RB, H), lambda i: (0, i, 0)),
        ],
        out_specs=pl.BlockSpec((RB, H), lambda i: (i, 0)),
        out_shape=jax.ShapeDtypeStruct((N, H), jnp.float32),
    )(relp, x, W_self.astype(jnp.float32), b_self.astype(jnp.float32), part)
    return out


# speedup vs baseline: 10.8527x; 10.8527x over previous
import functools

import jax
import jax.numpy as jnp
from jax import lax
from jax.experimental import pallas as pl
from jax.experimental.pallas import tpu as pltpu
from jax.experimental.pallas import tpu_sc as plsc

N = 10000
E = 320000
H = 128
R = 4

NC = 2
NS = 16
NW = NC * NS
CH = 128
EPW = ((E + NW * CH - 1) // (NW * CH)) * CH
EP = EPW * NW
PROWS = EP // 128
RB = 1000
NPAD = 10240
RPT = NPAD // NS


def _ymm_body(x_ref, w_ref, y_ref):
    y_ref[0] = jnp.dot(x_ref[...], w_ref[0], preferred_element_type=jnp.float32)


def _prep_body(relp_ref, src_ref, w_ref, gidx_ref, wm_ref):
    r0 = lax.broadcasted_iota(jnp.int32, (PROWS, 128), 0)
    c0 = lax.broadcasted_iota(jnp.int32, (PROWS, 128), 1)
    i = r0 * 128 + c0
    p0, p1, p2, p3, p4 = (relp_ref[k] for k in range(5))
    rel = ((i >= p1).astype(jnp.int32) + (i >= p2).astype(jnp.int32)
           + (i >= p3).astype(jnp.int32))
    valid = (i >= p0) & (i < p4)
    gidx_ref[...] = src_ref[...] + N * rel
    wm_ref[...] = w_ref[...] * valid.astype(jnp.float32)


def _comb_body(relp_ref, x_ref, ws_ref, bs_ref, p_ref, o_ref):
    act = [jnp.where(relp_ref[r + 1] > relp_ref[r], 1.0, 0.0).astype(jnp.float32)
           for r in range(R)]
    w_sum = (act[0] * ws_ref[0] + act[1] * ws_ref[1]
             + act[2] * ws_ref[2] + act[3] * ws_ref[3])
    b_sum = (act[0] * bs_ref[0] + act[1] * bs_ref[1]
             + act[2] * bs_ref[2] + act[3] * bs_ref[3])
    o_ref[...] = (jnp.dot(x_ref[...], w_sum, preferred_element_type=jnp.float32)
                  + b_sum[None, :] + p_ref[0] + p_ref[1])


def _sc_scatter(y_hbm, gidx_hbm, dst_hbm, w_hbm, z_hbm, out_hbm,
                acc, idx_v, dst_v, w_v, rows_v, sem):
    cid = lax.axis_index("c")
    sid = lax.axis_index("s")
    wid = sid * NC + cid
    pltpu.sync_copy(z_hbm, acc.at[pl.ds(sid * RPT, RPT)])
    plsc.subcore_barrier()

    ebase = wid * EPW

    dnums = lax.GatherDimensionNumbers(
        offset_dims=(), collapsed_slice_dims=(0,), start_index_map=(0,))

    def group(g, _):
        wvec = w_v[pl.ds(g * 16, 16)]
        for j in range(16):
            wk = lax.gather(wvec, jnp.full((16, 1), j, jnp.int32), dnums, (1,),
                            mode=lax.GatherScatterMode.PROMISE_IN_BOUNDS)
            k = g * 16 + j
            for jj in range(H // 16):
                sl = pl.ds(jj * 16, 16)
                rows_v[k, sl] = rows_v[k, sl] * wk
        return 0

    def chunk(c, _):
        b = pl.multiple_of(ebase + c * CH, CH)
        pltpu.sync_copy(gidx_hbm.at[pl.ds(b, CH)], idx_v)
        pltpu.sync_copy(dst_hbm.at[pl.ds(b, CH)], dst_v)
        pltpu.sync_copy(w_hbm.at[pl.ds(b, CH)], w_v)
        pltpu.async_copy(y_hbm.at[idx_v], rows_v, sem).wait()
        lax.fori_loop(0, CH // 16, group, 0)
        pltpu.sync_copy(rows_v, acc.at[dst_v], add=True)
        return 0

    lax.fori_loop(0, EPW // CH, chunk, 0)
    plsc.subcore_barrier()
    pltpu.sync_copy(acc.at[pl.ds(sid * RPT, RPT)],
                    out_hbm.at[cid, pl.ds(sid * RPT, RPT)])


def kernel(x, edge_index, edge_weight, rel_ptr, W_self, b_self, W_nei):
    x = x.astype(jnp.float32)
    src = edge_index[0].astype(jnp.int32)
    dst = edge_index[1].astype(jnp.int32)
    w = edge_weight.astype(jnp.float32)
    relp = rel_ptr.astype(jnp.int32)

    pad = EP - E
    src2 = jnp.pad(src, (0, pad)).reshape(PROWS, 128)
    w2 = jnp.pad(w, (0, pad)).reshape(PROWS, 128)
    dst_p = jnp.pad(dst, (0, pad))

    y = pl.pallas_call(
        _ymm_body,
        grid=(R, N // RB),
        in_specs=[
            pl.BlockSpec((RB, H), lambda r, i: (i, 0)),
            pl.BlockSpec((1, H, H), lambda r, i: (r, 0, 0)),
        ],
        out_specs=pl.BlockSpec((1, RB, H), lambda r, i: (r, i, 0)),
        out_shape=jax.ShapeDtypeStruct((R, N, H), jnp.float32),
    )(x, W_nei.astype(jnp.float32))
    y_flat = y.reshape(R * N, H)

    gidx2, wm2 = pl.pallas_call(
        _prep_body,
        in_specs=[
            pl.BlockSpec(memory_space=pltpu.SMEM),
            pl.BlockSpec((PROWS, 128), lambda: (0, 0)),
            pl.BlockSpec((PROWS, 128), lambda: (0, 0)),
        ],
        out_specs=[
            pl.BlockSpec((PROWS, 128), lambda: (0, 0)),
            pl.BlockSpec((PROWS, 128), lambda: (0, 0)),
        ],
        out_shape=[
            jax.ShapeDtypeStruct((PROWS, 128), jnp.int32),
            jax.ShapeDtypeStruct((PROWS, 128), jnp.float32),
        ],
    )(relp, src2, w2)
    gidx = gidx2.reshape(EP)
    wm = wm2.reshape(EP)

    zrows = jnp.zeros((RPT, H), jnp.float32)
    mesh = plsc.VectorSubcoreMesh(core_axis_name="c", subcore_axis_name="s",
                                  num_cores=NC, num_subcores=NS)
    part = functools.partial(
        pl.kernel,
        out_type=jax.ShapeDtypeStruct((NC, NPAD, H), jnp.float32),
        mesh=mesh,
        scratch_types=[
            pltpu.VMEM_SHARED((NPAD, H), jnp.float32),
            pltpu.VMEM((CH,), jnp.int32),
            pltpu.VMEM((CH,), jnp.int32),
            pltpu.VMEM((CH,), jnp.float32),
            pltpu.VMEM((CH, H), jnp.float32),
            pltpu.SemaphoreType.DMA,
        ],
    )(_sc_scatter)(y_flat, gidx, dst_p, wm, zrows)

    out = pl.pallas_call(
        _comb_body,
        grid=(N // RB,),
        in_specs=[
            pl.BlockSpec(memory_space=pltpu.SMEM),
            pl.BlockSpec((RB, H), lambda i: (i, 0)),
            pl.BlockSpec((R, H, H), lambda i: (0, 0, 0)),
            pl.BlockSpec((R, H), lambda i: (0, 0)),
            pl.BlockSpec((NC, RB, H), lambda i: (0, i, 0)),
        ],
        out_specs=pl.BlockSpec((RB, H), lambda i: (i, 0)),
        out_shape=jax.ShapeDtypeStruct((N, H), jnp.float32),
    )(relp, x, W_self.astype(jnp.float32), b_self.astype(jnp.float32), part)
    return out
